# baseline (speedup 1.0000x reference)
import functools

import jax
import jax.numpy as jnp
from jax import lax
from jax.experimental import pallas as pl
from jax.experimental.pallas import tpu as pltpu
from jax.experimental.pallas import tpu_sc as plsc

N_HEADS = 4
HEAD_DIM = 32
OUT_DIM = 128
PW = OUT_DIM

NCH = 5
E_CH = 64000
NW = 32
EDGES_PER_W = E_CH // NW
SUB = 80
IDX_ROWS_PER_W = EDGES_PER_W // SUB
G_SUB = 40
G_CHUNK = 200
G_SUBS = G_CHUNK // G_SUB
G_NCH = EDGES_PER_W // G_CHUNK
G_IDX_ROWS = EDGES_PER_W // G_SUB
S_CHUNK = 80
S_NCH = EDGES_PER_W // S_CHUNK


def _qd_body(dst_ref, wq1_ref, wq2_ref, bq_ref, phi_ref, o_ref):
    phi = phi_ref[...]
    te0 = jnp.concatenate([phi[:, :1], jnp.sin(phi[:, 1:])], axis=1)
    cvec = jnp.dot(te0, wq2_ref[...]) + bq_ref[...]
    o_ref[...] = jnp.dot(dst_ref[...], wq1_ref[...]) + cvec


def _fast_sin(x):
    inv_pi = 0.3183098861837907
    pi_hi = 3.140625
    pi_lo = 9.676535897932795e-4
    k = jnp.round(x * inv_pi)
    r = (x - k * pi_hi) - k * pi_lo
    r2 = r * r
    p = r * (1.0 + r2 * (-1.6666667e-1 + r2 * (8.3333333e-3
             + r2 * (-1.9841270e-4 + r2 * 2.7557319e-6))))
    odd = (k.astype(jnp.int32) & 1) == 1
    return jnp.where(odd, -p, p)


def _edge_body(src_ref, edge_ref, td_ref, qe_ref,
               w1hi_ref, w1lo_ref, w23hi_ref, w23lo_ref, bkv_ref, hm_ref,
               wt_ref, phi_ref, o_ref):
    scale = 1.0 / (HEAD_DIM ** 0.5)
    wt = td_ref[...] * wt_ref[...] + phi_ref[...]
    lane16 = lax.broadcasted_iota(jnp.int32, wt.shape, 1)
    te = jnp.where(lane16 == 0, wt, _fast_sin(wt))
    f32 = jnp.float32
    src = src_ref[...]
    src_hi = src.astype(jnp.bfloat16)
    src_lo = (src - src_hi.astype(f32)).astype(jnp.bfloat16)
    et = jnp.concatenate([edge_ref[...], te], axis=1)
    et_hi = et.astype(jnp.bfloat16)
    et_lo = (et - et_hi.astype(f32)).astype(jnp.bfloat16)
    kv = (jnp.dot(src_hi, w1hi_ref[...], preferred_element_type=f32)
          + jnp.dot(src_lo, w1hi_ref[...], preferred_element_type=f32)
          + jnp.dot(src_hi, w1lo_ref[...], preferred_element_type=f32)
          + jnp.dot(et_hi, w23hi_ref[...], preferred_element_type=f32)
          + jnp.dot(et_lo, w23hi_ref[...], preferred_element_type=f32)
          + jnp.dot(et_hi, w23lo_ref[...], preferred_element_type=f32)
          + bkv_ref[...])
    k = kv[:, :OUT_DIM]
    v = kv[:, OUT_DIM:]
    qk = qe_ref[...] * k
    s = jnp.dot(qk, hm_ref[...]) * scale
    a = jnp.exp(jnp.clip(s, -5.0, 5.0))
    w = jnp.sum(a, axis=1, keepdims=True) * (1.0 / N_HEADS)
    v0 = v[:, :1]
    e0 = (lax.broadcasted_iota(jnp.int32, v.shape, 1) == 0).astype(jnp.float32)
    o_ref[...] = w * (v - v0 + e0)


def _final_body(*refs):
    o_ref = refs[-1]
    g_ref = refs[-3]
    b_ref = refs[-2]
    acc = None
    for r in refs[:-3]:
        part = r[0] + r[1]
        acc = part if acc is None else acc + part
    norm = acc[:, :1]
    t = acc / (norm + 1e-8)
    lane = lax.broadcasted_iota(jnp.int32, t.shape, 1)
    t = jnp.where(lane == 0, 0.0, t)
    mu = jnp.mean(t, axis=1, keepdims=True)
    var = jnp.mean((t - mu) ** 2, axis=1, keepdims=True)
    o_ref[...] = (t - mu) / jnp.sqrt(var + 1e-5) * g_ref[...] + b_ref[...]


def _sc_mesh():
    return plsc.VectorSubcoreMesh(core_axis_name="c", subcore_axis_name="s")


def _gather_kernel(qd, ids4, ch):
    @functools.partial(
        pl.kernel,
        out_type=jax.ShapeDtypeStruct((E_CH, OUT_DIM), jnp.float32),
        mesh=_sc_mesh(),
        scratch_types=[
            pltpu.VMEM((G_IDX_ROWS, G_SUB), jnp.int32),
            pltpu.VMEM((G_CHUNK, OUT_DIM), jnp.float32),
            pltpu.VMEM((G_CHUNK, OUT_DIM), jnp.float32),
            pltpu.VMEM((G_CHUNK, OUT_DIM), jnp.float32),
            pltpu.VMEM((G_CHUNK, OUT_DIM), jnp.float32),
            pltpu.SemaphoreType.DMA,
            pltpu.SemaphoreType.DMA,
            pltpu.SemaphoreType.DMA,
            pltpu.SemaphoreType.DMA,
            pltpu.SemaphoreType.DMA,
            pltpu.SemaphoreType.DMA,
            pltpu.SemaphoreType.DMA,
            pltpu.SemaphoreType.DMA,
        ],
        name=f"qe_gather_{ch}",
    )
    def run(qd_hbm, ids_hbm, out_hbm, idx_v, rows_a, rows_b, rows_c, rows_d,
            sg0, sg1, sg2, sg3, sw0, sw1, sw2, sw3):
        wid = lax.axis_index("s") * 2 + lax.axis_index("c")
        base_e = pl.multiple_of(wid * EDGES_PER_W, 8)
        pltpu.sync_copy(ids_hbm.at[ch].at[wid], idx_v)

        bufs = (rows_a, rows_b, rows_c, rows_d)
        sgs = (sg0, sg1, sg2, sg3)
        sws = (sw0, sw1, sw2, sw3)
        depth = 4

        def issue_gathers(c):
            buf = bufs[c % depth]
            return [
                pltpu.async_copy(qd_hbm.at[idx_v.at[c * G_SUBS + j]],
                                 buf.at[pl.ds(j * G_SUB, G_SUB)],
                                 sgs[c % depth])
                for j in range(G_SUBS)
            ]

        gd = {c: issue_gathers(c) for c in range(depth - 1)}
        wd = {}
        for c in range(G_NCH):
            if c + depth - 1 < G_NCH:
                if c - 1 >= 0:
                    wd[c - 1].wait()
                gd[c + depth - 1] = issue_gathers(c + depth - 1)
            for d in gd[c]:
                d.wait()
            off = pl.multiple_of(base_e + c * G_CHUNK, 8)
            wd[c] = pltpu.async_copy(bufs[c % depth],
                                     out_hbm.at[pl.ds(off, G_CHUNK)],
                                     sws[c % depth])
        for c in range(max(0, G_NCH - depth), G_NCH):
            wd[c].wait()

    return run(qd, ids4)


def _scatter_kernel(p, ids4, zeros_nd, ch):
    n_dst = zeros_nd.shape[0]

    @functools.partial(
        pl.kernel,
        out_type=jax.ShapeDtypeStruct((2, n_dst, PW), jnp.float32),
        mesh=_sc_mesh(),
        scratch_types=[
            pltpu.VMEM((IDX_ROWS_PER_W, SUB), jnp.int32),
            pltpu.VMEM((S_CHUNK, PW), jnp.float32),
            pltpu.VMEM((S_CHUNK, PW), jnp.float32),
            pltpu.VMEM((S_CHUNK, PW), jnp.float32),
            pltpu.VMEM((S_CHUNK, PW), jnp.float32),
            pltpu.VMEM_SHARED((n_dst, PW), jnp.float32),
            pltpu.SemaphoreType.DMA,
            pltpu.SemaphoreType.DMA,
            pltpu.SemaphoreType.DMA,
            pltpu.SemaphoreType.DMA,
            pltpu.SemaphoreType.DMA,
            pltpu.SemaphoreType.DMA,
            pltpu.SemaphoreType.DMA,
            pltpu.SemaphoreType.DMA,
        ],
        name=f"seg_scatter_{ch}",
    )
    def run(p_hbm, ids_hbm, zero_hbm, out_hbm, idx_v, rows_a, rows_b, rows_c,
            rows_d, acc_sh, sl0, sl1, sl2, sl3, sa0, sa1, sa2, sa3):
        cid = lax.axis_index("c")
        sid = lax.axis_index("s")
        wid = sid * 2 + cid
        base_e = pl.multiple_of(wid * EDGES_PER_W, 8)

        @pl.when(sid == 0)
        def _():
            pltpu.sync_copy(zero_hbm, acc_sh)

        plsc.subcore_barrier()
        pltpu.sync_copy(ids_hbm.at[ch].at[wid], idx_v)

        bufs = (rows_a, rows_b, rows_c, rows_d)
        sls = (sl0, sl1, sl2, sl3)
        sas = (sa0, sa1, sa2, sa3)
        depth = 4

        def issue_load(c):
            off = pl.multiple_of(base_e + c * S_CHUNK, 8)
            return pltpu.async_copy(p_hbm.at[pl.ds(off, S_CHUNK)],
                                    bufs[c % depth], sls[c % depth])

        ld = {c: issue_load(c) for c in range(depth - 1)}
        ad = {}
        for c in range(S_NCH):
            if c + depth - 1 < S_NCH:
                if c - 1 >= 0:
                    ad[c - 1].wait()
                ld[c + depth - 1] = issue_load(c + depth - 1)
            ld[c].wait()
            ad[c] = pltpu.async_copy(bufs[c % depth], acc_sh.at[idx_v.at[c]],
                                     sas[c % depth], add=True)
        for c in range(max(0, S_NCH - depth), S_NCH):
            ad[c].wait()

        plsc.subcore_barrier()

        @pl.when(sid == 0)
        def _():
            pltpu.sync_copy(acc_sh, out_hbm.at[cid])

    return run(p, ids4, zeros_nd)


def kernel(src_feat, dst_feat, edge_feat, time_deltas, dst_ids, n_dst,
           query_time, w_t, phi_t, Wq_w, Wq_b, Wk_w, Wk_b, Wv_w, Wv_b,
           ln_g, ln_b):
    e = src_feat.shape[0]
    n_dst_s = dst_feat.shape[0]
    node_dim = src_feat.shape[1]
    edge_dim = edge_feat.shape[1]
    tdim = w_t.shape[0]

    phi2 = phi_t.reshape(1, tdim)
    wt2 = w_t.reshape(1, tdim)
    td2 = time_deltas.reshape(e, 1)
    ids4 = dst_ids.reshape(NCH, NW, IDX_ROWS_PER_W, SUB)
    ids4g = dst_ids.reshape(NCH, NW, G_IDX_ROWS, G_SUB)

    bq = 1000
    qd = pl.pallas_call(
        _qd_body,
        grid=(n_dst_s // bq,),
        in_specs=[
            pl.BlockSpec((bq, node_dim), lambda i: (i, 0)),
            pl.BlockSpec((node_dim, OUT_DIM), lambda i: (0, 0)),
            pl.BlockSpec((tdim, OUT_DIM), lambda i: (0, 0)),
            pl.BlockSpec((1, OUT_DIM), lambda i: (0, 0)),
            pl.BlockSpec((1, tdim), lambda i: (0, 0)),
        ],
        out_specs=pl.BlockSpec((bq, OUT_DIM), lambda i: (i, 0)),
        out_shape=jax.ShapeDtypeStruct((n_dst_s, OUT_DIM), jnp.float32),
    )(dst_feat, Wq_w[:node_dim], Wq_w[node_dim:], Wq_b.reshape(1, OUT_DIM),
      phi2)

    wkv1 = jnp.concatenate([Wk_w[:node_dim], Wv_w[:node_dim]], axis=1)
    w1hi = wkv1.astype(jnp.bfloat16)
    w1lo = (wkv1 - w1hi.astype(jnp.float32)).astype(jnp.bfloat16)
    wkv23 = jnp.concatenate([Wk_w[node_dim:], Wv_w[node_dim:]],
                            axis=1)
    w23hi = wkv23.astype(jnp.bfloat16)
    w23lo = (wkv23 - w23hi.astype(jnp.float32)).astype(jnp.bfloat16)
    bkv = jnp.concatenate([Wk_b, Wv_b]).reshape(1, 2 * OUT_DIM)
    hm = (lax.broadcasted_iota(jnp.int32, (OUT_DIM, N_HEADS), 0) // HEAD_DIM
          == lax.broadcasted_iota(jnp.int32, (OUT_DIM, N_HEADS), 1)
          ).astype(jnp.float32)
    zeros_nd = jnp.zeros((n_dst_s, PW), jnp.float32)
    be = 2000
    blocks_per_ch = E_CH // be
    qes = [_gather_kernel(qd, ids4g, ch) for ch in range(NCH)]
    partials = []
    for ch in range(NCH):
        qe = qes[ch]
        em = functools.partial(lambda i, c: (i + c * blocks_per_ch, 0), c=ch)
        p = pl.pallas_call(
            _edge_body,
            grid=(blocks_per_ch,),
            in_specs=[
                pl.BlockSpec((be, node_dim), em),
                pl.BlockSpec((be, edge_dim), em),
                pl.BlockSpec((be, 1), em),
                pl.BlockSpec((be, OUT_DIM), lambda i: (i, 0)),
                pl.BlockSpec((node_dim, 2 * OUT_DIM), lambda i: (0, 0)),
                pl.BlockSpec((node_dim, 2 * OUT_DIM), lambda i: (0, 0)),
                pl.BlockSpec((edge_dim + tdim, 2 * OUT_DIM),
                             lambda i: (0, 0)),
                pl.BlockSpec((edge_dim + tdim, 2 * OUT_DIM),
                             lambda i: (0, 0)),
                pl.BlockSpec((1, 2 * OUT_DIM), lambda i: (0, 0)),
                pl.BlockSpec((OUT_DIM, N_HEADS), lambda i: (0, 0)),
                pl.BlockSpec((1, tdim), lambda i: (0, 0)),
                pl.BlockSpec((1, tdim), lambda i: (0, 0)),
            ],
            out_specs=pl.BlockSpec((be, PW), lambda i: (i, 0)),
            out_shape=jax.ShapeDtypeStruct((E_CH, PW), jnp.float32),
            name=f"edge_pipe_{ch}",
        )(src_feat, edge_feat, td2, qe, w1hi, w1lo, w23hi, w23lo, bkv, hm, wt2,
          phi2)
        partials.append(_scatter_kernel(p, ids4, zeros_nd, ch))

    bf = 1000
    out = pl.pallas_call(
        _final_body,
        grid=(n_dst_s // bf,),
        in_specs=[pl.BlockSpec((2, bf, PW), lambda i: (0, i, 0))] * NCH + [
            pl.BlockSpec((1, OUT_DIM), lambda i: (0, 0)),
            pl.BlockSpec((1, OUT_DIM), lambda i: (0, 0)),
        ],
        out_specs=pl.BlockSpec((bf, OUT_DIM), lambda i: (i, 0)),
        out_shape=jax.ShapeDtypeStruct((n_dst_s, OUT_DIM), jnp.float32),
    )(*partials, ln_g.reshape(1, OUT_DIM), ln_b.reshape(1, OUT_DIM))

    return out


# speedup vs baseline: 1.0035x; 1.0035x over previous
import functools

import jax
import jax.numpy as jnp
from jax import lax
from jax.experimental import pallas as pl
from jax.experimental.pallas import tpu as pltpu
from jax.experimental.pallas import tpu_sc as plsc

N_HEADS = 4
HEAD_DIM = 32
OUT_DIM = 128
PW = OUT_DIM

NCH = 5
E_CH = 64000
NW = 32
EDGES_PER_W = E_CH // NW
SUB = 80
IDX_ROWS_PER_W = EDGES_PER_W // SUB
G_SUB = 80
G_CHUNK = 400
G_SUBS = G_CHUNK // G_SUB
G_NCH = EDGES_PER_W // G_CHUNK
G_IDX_ROWS = EDGES_PER_W // G_SUB
S_CHUNK = 80
S_NCH = EDGES_PER_W // S_CHUNK


def _qd_body(dst_ref, wq1_ref, wq2_ref, bq_ref, phi_ref, o_ref):
    phi = phi_ref[...]
    te0 = jnp.concatenate([phi[:, :1], jnp.sin(phi[:, 1:])], axis=1)
    cvec = jnp.dot(te0, wq2_ref[...]) + bq_ref[...]
    o_ref[...] = jnp.dot(dst_ref[...], wq1_ref[...]) + cvec


def _fast_sin(x):
    inv_pi = 0.3183098861837907
    pi_hi = 3.140625
    pi_lo = 9.676535897932795e-4
    k = jnp.round(x * inv_pi)
    r = (x - k * pi_hi) - k * pi_lo
    r2 = r * r
    p = r * (1.0 + r2 * (-1.6666667e-1 + r2 * (8.3333333e-3
             + r2 * (-1.9841270e-4 + r2 * 2.7557319e-6))))
    odd = (k.astype(jnp.int32) & 1) == 1
    return jnp.where(odd, -p, p)


def _edge_body(src_ref, edge_ref, td_ref, qe_ref,
               w1hi_ref, w1lo_ref, w23hi_ref, w23lo_ref, bkv_ref, hm_ref,
               wt_ref, phi_ref, o_ref):
    scale = 1.0 / (HEAD_DIM ** 0.5)
    wt = td_ref[...] * wt_ref[...] + phi_ref[...]
    lane16 = lax.broadcasted_iota(jnp.int32, wt.shape, 1)
    te = jnp.where(lane16 == 0, wt, _fast_sin(wt))
    f32 = jnp.float32
    src = src_ref[...]
    src_hi = src.astype(jnp.bfloat16)
    src_lo = (src - src_hi.astype(f32)).astype(jnp.bfloat16)
    et = jnp.concatenate([edge_ref[...], te], axis=1)
    et_hi = et.astype(jnp.bfloat16)
    et_lo = (et - et_hi.astype(f32)).astype(jnp.bfloat16)
    kv = (jnp.dot(src_hi, w1hi_ref[...], preferred_element_type=f32)
          + jnp.dot(src_lo, w1hi_ref[...], preferred_element_type=f32)
          + jnp.dot(src_hi, w1lo_ref[...], preferred_element_type=f32)
          + jnp.dot(et_hi, w23hi_ref[...], preferred_element_type=f32)
          + jnp.dot(et_lo, w23hi_ref[...], preferred_element_type=f32)
          + jnp.dot(et_hi, w23lo_ref[...], preferred_element_type=f32)
          + bkv_ref[...])
    k = kv[:, :OUT_DIM]
    v = kv[:, OUT_DIM:]
    qk = qe_ref[...] * k
    s = jnp.dot(qk, hm_ref[...]) * scale
    a = jnp.exp(jnp.clip(s, -5.0, 5.0))
    w = jnp.sum(a, axis=1, keepdims=True) * (1.0 / N_HEADS)
    v0 = v[:, :1]
    e0 = (lax.broadcasted_iota(jnp.int32, v.shape, 1) == 0).astype(jnp.float32)
    o_ref[...] = w * (v - v0 + e0)


def _final_body(*refs):
    o_ref = refs[-1]
    g_ref = refs[-3]
    b_ref = refs[-2]
    acc = None
    for r in refs[:-3]:
        part = r[0] + r[1]
        acc = part if acc is None else acc + part
    norm = acc[:, :1]
    t = acc / (norm + 1e-8)
    lane = lax.broadcasted_iota(jnp.int32, t.shape, 1)
    t = jnp.where(lane == 0, 0.0, t)
    mu = jnp.mean(t, axis=1, keepdims=True)
    var = jnp.mean((t - mu) ** 2, axis=1, keepdims=True)
    o_ref[...] = (t - mu) / jnp.sqrt(var + 1e-5) * g_ref[...] + b_ref[...]


def _sc_mesh():
    return plsc.VectorSubcoreMesh(core_axis_name="c", subcore_axis_name="s")


def _gather_kernel(qd, ids4, ch):
    @functools.partial(
        pl.kernel,
        out_type=jax.ShapeDtypeStruct((E_CH, OUT_DIM), jnp.float32),
        mesh=_sc_mesh(),
        scratch_types=[
            pltpu.VMEM((G_IDX_ROWS, G_SUB), jnp.int32),
            pltpu.VMEM((G_CHUNK, OUT_DIM), jnp.float32),
            pltpu.VMEM((G_CHUNK, OUT_DIM), jnp.float32),
            pltpu.SemaphoreType.DMA,
            pltpu.SemaphoreType.DMA,
            pltpu.SemaphoreType.DMA,
            pltpu.SemaphoreType.DMA,
        ],
        name=f"qe_gather_{ch}",
    )
    def run(qd_hbm, ids_hbm, out_hbm, idx_v, rows_a, rows_b,
            sg0, sg1, sw0, sw1):
        wid = lax.axis_index("s") * 2 + lax.axis_index("c")
        base_e = pl.multiple_of(wid * EDGES_PER_W, 8)
        pltpu.sync_copy(ids_hbm.at[ch].at[wid], idx_v)

        bufs = (rows_a, rows_b)
        sgs = (sg0, sg1)
        sws = (sw0, sw1)
        depth = 2

        def issue_gathers(c):
            buf = bufs[c % depth]
            return [
                pltpu.async_copy(qd_hbm.at[idx_v.at[c * G_SUBS + j]],
                                 buf.at[pl.ds(j * G_SUB, G_SUB)],
                                 sgs[c % depth])
                for j in range(G_SUBS)
            ]

        gd = {c: issue_gathers(c) for c in range(depth - 1)}
        wd = {}
        for c in range(G_NCH):
            if c + depth - 1 < G_NCH:
                if c - 1 >= 0:
                    wd[c - 1].wait()
                gd[c + depth - 1] = issue_gathers(c + depth - 1)
            for d in gd[c]:
                d.wait()
            off = pl.multiple_of(base_e + c * G_CHUNK, 8)
            wd[c] = pltpu.async_copy(bufs[c % depth],
                                     out_hbm.at[pl.ds(off, G_CHUNK)],
                                     sws[c % depth])
        for c in range(max(0, G_NCH - depth), G_NCH):
            wd[c].wait()

    return run(qd, ids4)


def _scatter_kernel(p, ids4, zeros_nd, ch):
    n_dst = zeros_nd.shape[0]

    @functools.partial(
        pl.kernel,
        out_type=jax.ShapeDtypeStruct((2, n_dst, PW), jnp.float32),
        mesh=_sc_mesh(),
        scratch_types=[
            pltpu.VMEM((IDX_ROWS_PER_W, SUB), jnp.int32),
            pltpu.VMEM((S_CHUNK, PW), jnp.float32),
            pltpu.VMEM((S_CHUNK, PW), jnp.float32),
            pltpu.VMEM((S_CHUNK, PW), jnp.float32),
            pltpu.VMEM((S_CHUNK, PW), jnp.float32),
            pltpu.VMEM_SHARED((n_dst, PW), jnp.float32),
            pltpu.SemaphoreType.DMA,
            pltpu.SemaphoreType.DMA,
            pltpu.SemaphoreType.DMA,
            pltpu.SemaphoreType.DMA,
            pltpu.SemaphoreType.DMA,
            pltpu.SemaphoreType.DMA,
            pltpu.SemaphoreType.DMA,
            pltpu.SemaphoreType.DMA,
        ],
        name=f"seg_scatter_{ch}",
    )
    def run(p_hbm, ids_hbm, zero_hbm, out_hbm, idx_v, rows_a, rows_b, rows_c,
            rows_d, acc_sh, sl0, sl1, sl2, sl3, sa0, sa1, sa2, sa3):
        cid = lax.axis_index("c")
        sid = lax.axis_index("s")
        wid = sid * 2 + cid
        base_e = pl.multiple_of(wid * EDGES_PER_W, 8)

        @pl.when(sid == 0)
        def _():
            pltpu.sync_copy(zero_hbm, acc_sh)

        plsc.subcore_barrier()
        pltpu.sync_copy(ids_hbm.at[ch].at[wid], idx_v)

        bufs = (rows_a, rows_b, rows_c, rows_d)
        sls = (sl0, sl1, sl2, sl3)
        sas = (sa0, sa1, sa2, sa3)
        depth = 4

        def issue_load(c):
            off = pl.multiple_of(base_e + c * S_CHUNK, 8)
            return pltpu.async_copy(p_hbm.at[pl.ds(off, S_CHUNK)],
                                    bufs[c % depth], sls[c % depth])

        ld = {c: issue_load(c) for c in range(depth - 1)}
        ad = {}
        for c in range(S_NCH):
            if c + depth - 1 < S_NCH:
                if c - 1 >= 0:
                    ad[c - 1].wait()
                ld[c + depth - 1] = issue_load(c + depth - 1)
            ld[c].wait()
            ad[c] = pltpu.async_copy(bufs[c % depth], acc_sh.at[idx_v.at[c]],
                                     sas[c % depth], add=True)
        for c in range(max(0, S_NCH - depth), S_NCH):
            ad[c].wait()

        plsc.subcore_barrier()

        @pl.when(sid == 0)
        def _():
            pltpu.sync_copy(acc_sh, out_hbm.at[cid])

    return run(p, ids4, zeros_nd)


def kernel(src_feat, dst_feat, edge_feat, time_deltas, dst_ids, n_dst,
           query_time, w_t, phi_t, Wq_w, Wq_b, Wk_w, Wk_b, Wv_w, Wv_b,
           ln_g, ln_b):
    e = src_feat.shape[0]
    n_dst_s = dst_feat.shape[0]
    node_dim = src_feat.shape[1]
    edge_dim = edge_feat.shape[1]
    tdim = w_t.shape[0]

    phi2 = phi_t.reshape(1, tdim)
    wt2 = w_t.reshape(1, tdim)
    td2 = time_deltas.reshape(e, 1)
    ids4 = dst_ids.reshape(NCH, NW, IDX_ROWS_PER_W, SUB)
    ids4g = dst_ids.reshape(NCH, NW, G_IDX_ROWS, G_SUB)

    bq = 1000
    qd = pl.pallas_call(
        _qd_body,
        grid=(n_dst_s // bq,),
        in_specs=[
            pl.BlockSpec((bq, node_dim), lambda i: (i, 0)),
            pl.BlockSpec((node_dim, OUT_DIM), lambda i: (0, 0)),
            pl.BlockSpec((tdim, OUT_DIM), lambda i: (0, 0)),
            pl.BlockSpec((1, OUT_DIM), lambda i: (0, 0)),
            pl.BlockSpec((1, tdim), lambda i: (0, 0)),
        ],
        out_specs=pl.BlockSpec((bq, OUT_DIM), lambda i: (i, 0)),
        out_shape=jax.ShapeDtypeStruct((n_dst_s, OUT_DIM), jnp.float32),
    )(dst_feat, Wq_w[:node_dim], Wq_w[node_dim:], Wq_b.reshape(1, OUT_DIM),
      phi2)

    wkv1 = jnp.concatenate([Wk_w[:node_dim], Wv_w[:node_dim]], axis=1)
    w1hi = wkv1.astype(jnp.bfloat16)
    w1lo = (wkv1 - w1hi.astype(jnp.float32)).astype(jnp.bfloat16)
    wkv23 = jnp.concatenate([Wk_w[node_dim:], Wv_w[node_dim:]],
                            axis=1)
    w23hi = wkv23.astype(jnp.bfloat16)
    w23lo = (wkv23 - w23hi.astype(jnp.float32)).astype(jnp.bfloat16)
    bkv = jnp.concatenate([Wk_b, Wv_b]).reshape(1, 2 * OUT_DIM)
    hm = (lax.broadcasted_iota(jnp.int32, (OUT_DIM, N_HEADS), 0) // HEAD_DIM
          == lax.broadcasted_iota(jnp.int32, (OUT_DIM, N_HEADS), 1)
          ).astype(jnp.float32)
    zeros_nd = jnp.zeros((n_dst_s, PW), jnp.float32)
    be = 2000
    blocks_per_ch = E_CH // be
    qes = [_gather_kernel(qd, ids4g, ch) for ch in range(NCH)]
    partials = []
    for ch in range(NCH):
        qe = qes[ch]
        em = functools.partial(lambda i, c: (i + c * blocks_per_ch, 0), c=ch)
        p = pl.pallas_call(
            _edge_body,
            grid=(blocks_per_ch,),
            in_specs=[
                pl.BlockSpec((be, node_dim), em),
                pl.BlockSpec((be, edge_dim), em),
                pl.BlockSpec((be, 1), em),
                pl.BlockSpec((be, OUT_DIM), lambda i: (i, 0)),
                pl.BlockSpec((node_dim, 2 * OUT_DIM), lambda i: (0, 0)),
                pl.BlockSpec((node_dim, 2 * OUT_DIM), lambda i: (0, 0)),
                pl.BlockSpec((edge_dim + tdim, 2 * OUT_DIM),
                             lambda i: (0, 0)),
                pl.BlockSpec((edge_dim + tdim, 2 * OUT_DIM),
                             lambda i: (0, 0)),
                pl.BlockSpec((1, 2 * OUT_DIM), lambda i: (0, 0)),
                pl.BlockSpec((OUT_DIM, N_HEADS), lambda i: (0, 0)),
                pl.BlockSpec((1, tdim), lambda i: (0, 0)),
                pl.BlockSpec((1, tdim), lambda i: (0, 0)),
            ],
            out_specs=pl.BlockSpec((be, PW), lambda i: (i, 0)),
            out_shape=jax.ShapeDtypeStruct((E_CH, PW), jnp.float32),
            name=f"edge_pipe_{ch}",
        )(src_feat, edge_feat, td2, qe, w1hi, w1lo, w23hi, w23lo, bkv, hm, wt2,
          phi2)
        partials.append(_scatter_kernel(p, ids4, zeros_nd, ch))

    bf = 1000
    out = pl.pallas_call(
        _final_body,
        grid=(n_dst_s // bf,),
        in_specs=[pl.BlockSpec((2, bf, PW), lambda i: (0, i, 0))] * NCH + [
            pl.BlockSpec((1, OUT_DIM), lambda i: (0, 0)),
            pl.BlockSpec((1, OUT_DIM), lambda i: (0, 0)),
        ],
        out_specs=pl.BlockSpec((bf, OUT_DIM), lambda i: (i, 0)),
        out_shape=jax.ShapeDtypeStruct((n_dst_s, OUT_DIM), jnp.float32),
    )(*partials, ln_g.reshape(1, OUT_DIM), ln_b.reshape(1, OUT_DIM))

    return out
